# baseline (speedup 1.0000x reference)
import functools

import jax
import jax.numpy as jnp
from jax import lax
from jax.experimental import pallas as pl
from jax.experimental.pallas import tpu as pltpu
from jax.experimental.pallas import tpu_sc as plsc

_GLOBAL_MEAN = 3.5
_NC = 2
_NS = 16
_NW = _NC * _NS
_L = 16
_CHUNK = 128


@functools.partial(jax.jit, static_argnames=())
def _gmf_sc(uid, mid, user_emb, movie_emb, user_bias, movie_bias, w_bcast):
    B = uid.shape[0]
    D = user_emb.shape[1]
    BPW = B // _NW
    NCHUNK = BPW // _CHUNK
    G = BPW // _L

    mesh = plsc.VectorSubcoreMesh(core_axis_name="c", subcore_axis_name="s")

    @functools.partial(
        pl.kernel,
        mesh=mesh,
        out_type=jax.ShapeDtypeStruct((B,), jnp.float32),
        compiler_params=pltpu.CompilerParams(
            needs_layout_passes=False, use_tc_tiling_on_sc=False),
        scratch_types=[
            pltpu.VMEM((NCHUNK, _CHUNK), jnp.int32),
            pltpu.VMEM((NCHUNK, _CHUNK), jnp.int32),
            pltpu.VMEM((BPW, D), jnp.float32),
            pltpu.VMEM((BPW, D), jnp.float32),
            pltpu.VMEM((BPW, 1), jnp.float32),
            pltpu.VMEM((BPW, 1), jnp.float32),
            pltpu.VMEM((D, _L), jnp.float32),
            pltpu.VMEM((BPW,), jnp.float32),
            pltpu.SemaphoreType.DMA,
        ],
    )
    def body(uid_hbm, mid_hbm, uemb_hbm, memb_hbm, ub_hbm, mb_hbm, w_hbm,
             out_hbm, uidx, midx, urows, mrows, ubv, mbv, wv, ob, sem):
        wid = lax.axis_index("s") * _NC + lax.axis_index("c")
        base = wid * BPW

        pltpu.sync_copy(w_hbm, wv)
        for c in range(NCHUNK):
            pltpu.sync_copy(uid_hbm.at[pl.ds(base + c * _CHUNK, _CHUNK)],
                            uidx.at[c])
            pltpu.sync_copy(mid_hbm.at[pl.ds(base + c * _CHUNK, _CHUNK)],
                            midx.at[c])

        copies = []
        for c in range(NCHUNK):
            sl = pl.ds(c * _CHUNK, _CHUNK)
            copies.append(pltpu.async_copy(
                uemb_hbm.at[uidx.at[c]], urows.at[sl], sem))
            copies.append(pltpu.async_copy(
                memb_hbm.at[midx.at[c]], mrows.at[sl], sem))
            copies.append(pltpu.async_copy(
                ub_hbm.at[uidx.at[c]], ubv.at[sl], sem))
            copies.append(pltpu.async_copy(
                mb_hbm.at[midx.at[c]], mbv.at[sl], sem))
        for cp in copies:
            cp.wait()

        zeros16 = jnp.zeros((_L,), jnp.int32)
        lane = lax.iota(jnp.int32, _L)

        def group_body(g, carry):
            bidx = lane + g * _L
            acc = jnp.zeros((_L,), jnp.float32)
            for d in range(D):
                dsplat = jnp.full((_L,), d, jnp.int32)
                uv = plsc.load_gather(urows, [bidx, dsplat])
                mv = plsc.load_gather(mrows, [bidx, dsplat])
                acc = acc + uv * mv * wv[d, :]
            bu = plsc.load_gather(ubv, [bidx, zeros16])
            bm = plsc.load_gather(mbv, [bidx, zeros16])
            ob[pl.ds(g * _L, _L)] = acc + bu + bm + _GLOBAL_MEAN
            return carry

        lax.fori_loop(0, G, group_body, 0)

        pltpu.sync_copy(ob, out_hbm.at[pl.ds(base, BPW)])

    return body(uid, mid, user_emb, movie_emb, user_bias, movie_bias, w_bcast)


def kernel(user_ids, movie_ids, user_emb, movie_emb, user_bias, movie_bias,
           affine_w):
    uid = user_ids.astype(jnp.int32)
    mid = movie_ids.astype(jnp.int32)
    D = user_emb.shape[1]
    w_bcast = jnp.broadcast_to(affine_w.reshape(D, 1), (D, _L)).astype(
        jnp.float32)
    return _gmf_sc(uid, mid, user_emb, movie_emb, user_bias, movie_bias,
                   w_bcast)
